# baseline (speedup 1.0000x reference)
import functools

import jax
import jax.numpy as jnp
from jax import lax
from jax.experimental import pallas as pl
from jax.experimental.pallas import tpu as pltpu
from jax.experimental.pallas import tpu_sc as plsc

HOPS = 3
EMB = 32
B, M, S = 1024, 50, 20
NWORDS = 100000
NTABLES = HOPS + 1
FEMB = NTABLES * EMB
NW = 32
G = 16
ROWS = G * S
NLANE = FEMB // 16

NSPLIT = 4
BSLICE = B // NSPLIT
SEG_SL = BSLICE * M
SEG_PER_TILE_SL = SEG_SL // NW
NCHUNK_SL = SEG_PER_TILE_SL // G


def _tc_fuse_tables(C0, C1, C2, C3):
  BLK = 5000

  def body(c0, c1, c2, c3, o):
    o[...] = jnp.concatenate([c0[...], c1[...], c2[...], c3[...]], axis=1)

  spec = pl.BlockSpec((BLK, EMB), lambda i: (i, 0))
  return pl.pallas_call(
      body,
      grid=(NWORDS // BLK,),
      in_specs=[spec, spec, spec, spec],
      out_specs=pl.BlockSpec((BLK, FEMB), lambda i: (i, 0)),
      out_shape=jax.ShapeDtypeStruct((NWORDS, FEMB), jnp.float32),
  )(C0, C1, C2, C3)


def _sc_gather_sum(idx2d, T4):
  mesh = plsc.VectorSubcoreMesh(core_axis_name="c", subcore_axis_name="s")

  @functools.partial(
      pl.kernel,
      out_type=jax.ShapeDtypeStruct((SEG_SL, FEMB), jnp.float32),
      mesh=mesh,
      compiler_params=pltpu.CompilerParams(use_tc_tiling_on_sc=False),
      scratch_types=[
          pltpu.VMEM((NCHUNK_SL, ROWS), jnp.int32),
          pltpu.VMEM((2, ROWS, FEMB), jnp.float32),
          pltpu.VMEM((2, G, FEMB), jnp.float32),
          pltpu.SemaphoreType.DMA,
          pltpu.SemaphoreType.DMA,
          pltpu.SemaphoreType.DMA,
          pltpu.SemaphoreType.DMA,
      ],
  )
  def k(idx_hbm, t4, out_hbm, idx_v, rows_v, e_v, g0, g1, w0, w1):
    wid = lax.axis_index("s") * 2 + lax.axis_index("c")
    gsem = [g0, g1]
    wsem = [w0, w1]

    pltpu.sync_copy(idx_hbm.at[wid], idx_v)

    def fire(c, par):
      pltpu.async_copy(t4.at[idx_v.at[c]], rows_v.at[par], gsem[par])

    def wait_gather(c, par):
      pltpu.make_async_copy(
          t4.at[idx_v.at[c]], rows_v.at[par], gsem[par]).wait()

    def wait_write(par):
      pltpu.make_async_copy(
          e_v.at[par], out_hbm.at[pl.ds(0, G)], wsem[par]).wait()

    def sum_chunk(par):
      rows = rows_v.at[par]
      e = e_v.at[par]

      def seg_body(g, _):
        r0 = g * S
        acc = [jnp.zeros((16,), jnp.float32) for _ in range(NLANE)]
        for s in range(S):
          for j in range(NLANE):
            acc[j] = acc[j] + rows[r0 + s, 16 * j:16 * j + 16]
        for j in range(NLANE):
          e[g, 16 * j:16 * j + 16] = acc[j]
        return 0

      lax.fori_loop(0, G, seg_body, 0)

    fire(0, 0)

    def pair_body(cp, _):
      for par in (0, 1):
        c = cp * 2 + par

        @pl.when(c < NCHUNK_SL - 1)
        def _():
          fire(c + 1, 1 - par)

        wait_gather(c, par)

        @pl.when(cp > 0)
        def _():
          wait_write(par)

        sum_chunk(par)
        seg_base = wid * SEG_PER_TILE_SL + c * G
        pltpu.async_copy(
            e_v.at[par], out_hbm.at[pl.ds(seg_base, G)], wsem[par])
      return 0

    lax.fori_loop(0, NCHUNK_SL // 2, pair_body, 0)
    if NCHUNK_SL % 2:
      c = NCHUNK_SL - 1
      wait_gather(c, 0)
      wait_write(0)
      sum_chunk(0)
      seg_base = wid * SEG_PER_TILE_SL + c * G
      pltpu.async_copy(
          e_v.at[0], out_hbm.at[pl.ds(seg_base, G)], wsem[0])
    wait_write(0)
    wait_write(1)

  return k(idx2d, T4)


def _sc_gather_sum_sep(idx2d, C0, C1, C2, C3):
  mesh = plsc.VectorSubcoreMesh(core_axis_name="c", subcore_axis_name="s")

  @functools.partial(
      pl.kernel,
      out_type=jax.ShapeDtypeStruct((SEG_SL, FEMB), jnp.float32),
      mesh=mesh,
      compiler_params=pltpu.CompilerParams(use_tc_tiling_on_sc=False),
      scratch_types=[
          pltpu.VMEM((NCHUNK_SL, ROWS), jnp.int32),
          pltpu.VMEM((2, NTABLES, ROWS, EMB), jnp.float32),
          pltpu.VMEM((2, G, FEMB), jnp.float32),
          pltpu.SemaphoreType.DMA,
          pltpu.SemaphoreType.DMA,
          pltpu.SemaphoreType.DMA,
          pltpu.SemaphoreType.DMA,
      ],
  )
  def k(idx_hbm, t0, t1, t2, t3, out_hbm, idx_v, rows_v, e_v, g0, g1, w0, w1):
    wid = lax.axis_index("s") * 2 + lax.axis_index("c")
    tabs = [t0, t1, t2, t3]
    gsem = [g0, g1]
    wsem = [w0, w1]

    pltpu.sync_copy(idx_hbm.at[wid], idx_v)

    def fire(c, par):
      for h in range(NTABLES):
        pltpu.async_copy(
            tabs[h].at[idx_v.at[c]], rows_v.at[par, h], gsem[par])

    def wait_gather(c, par):
      for h in range(NTABLES):
        pltpu.make_async_copy(
            tabs[h].at[idx_v.at[c]], rows_v.at[par, h], gsem[par]).wait()

    def wait_write(par):
      pltpu.make_async_copy(
          e_v.at[par], out_hbm.at[pl.ds(0, G)], wsem[par]).wait()

    def sum_chunk(par):
      e = e_v.at[par]

      def seg_body(g, _):
        r0 = g * S
        acc = [jnp.zeros((16,), jnp.float32) for _ in range(NLANE)]
        for s in range(S):
          for j in range(NLANE):
            acc[j] = acc[j] + rows_v[par, j // 2, r0 + s,
                                     16 * (j % 2):16 * (j % 2) + 16]
        for j in range(NLANE):
          e[g, 16 * j:16 * j + 16] = acc[j]
        return 0

      lax.fori_loop(0, G, seg_body, 0)

    fire(0, 0)

    def pair_body(cp, _):
      for par in (0, 1):
        c = cp * 2 + par

        @pl.when(c < NCHUNK_SL - 1)
        def _():
          fire(c + 1, 1 - par)

        wait_gather(c, par)

        @pl.when(cp > 0)
        def _():
          wait_write(par)

        sum_chunk(par)
        seg_base = wid * SEG_PER_TILE_SL + c * G
        pltpu.async_copy(
            e_v.at[par], out_hbm.at[pl.ds(seg_base, G)], wsem[par])
      return 0

    lax.fori_loop(0, NCHUNK_SL // 2, pair_body, 0)
    if NCHUNK_SL % 2:
      c = NCHUNK_SL - 1
      wait_gather(c, 0)
      wait_write(0)
      sum_chunk(0)
      seg_base = wid * SEG_PER_TILE_SL + c * G
      pltpu.async_copy(
          e_v.at[0], out_hbm.at[pl.ds(seg_base, G)], wsem[0])
    wait_write(0)
    wait_write(1)

  return k(idx2d, C0, C1, C2, C3)


def _tc_attention(e_all):
  BB = 128

  def body(e_ref, o_ref):
    e = e_ref[...]
    q = jnp.zeros((BB, EMB), jnp.float32)
    o2 = None
    for h in range(HOPS):
      mm = e[:, :, EMB * h:EMB * h + EMB]
      p = jnp.sum(mm * q[:, None, :], axis=2)
      p = p - jnp.max(p, axis=1, keepdims=True)
      a = jnp.exp(p)
      a = a / jnp.sum(a, axis=1, keepdims=True)
      c = e[:, :, EMB * (h + 1):EMB * (h + 1) + EMB]
      o2 = jnp.sum(c * a[:, :, None], axis=1)
      q = q + o2
    o_ref[...] = o2

  return pl.pallas_call(
      body,
      grid=(BSLICE // BB,),
      in_specs=[pl.BlockSpec((BB, M, FEMB), lambda i: (i, 0, 0))],
      out_specs=pl.BlockSpec((BB, EMB), lambda i: (i, 0)),
      out_shape=jax.ShapeDtypeStruct((BSLICE, EMB), jnp.float32),
  )(e_all)


def kernel(context, C0, C1, C2, C3):
  idx_all = context.reshape(NSPLIT, NW, NCHUNK_SL, ROWS).astype(jnp.int32)
  T4 = jnp.concatenate([C0, C1, C2, C3], axis=1)
  outs = []
  for s in range(NSPLIT):
    if s == 0:
      e4 = _sc_gather_sum_sep(idx_all[s], C0, C1, C2, C3)
    else:
      e4 = _sc_gather_sum(idx_all[s], T4)
    outs.append(_tc_attention(e4.reshape(BSLICE, M, FEMB)))
  return jnp.concatenate(outs, axis=0)


# speedup vs baseline: 1.1247x; 1.1247x over previous
import functools

import jax
import jax.numpy as jnp
from jax import lax
from jax.experimental import pallas as pl
from jax.experimental.pallas import tpu as pltpu
from jax.experimental.pallas import tpu_sc as plsc

HOPS = 3
EMB = 32
B, M, S = 1024, 50, 20
NWORDS = 100000
NTABLES = HOPS + 1
FEMB = NTABLES * EMB
NW = 32
G = 16
ROWS = G * S
NLANE = FEMB // 16

NSPLIT = 4
BSLICE = B // NSPLIT
SEG_SL = BSLICE * M
SEG_PER_TILE_SL = SEG_SL // NW
NCHUNK_SL = SEG_PER_TILE_SL // G


def _sc_gather_sum(idx2d, T4):
  mesh = plsc.VectorSubcoreMesh(core_axis_name="c", subcore_axis_name="s")

  @functools.partial(
      pl.kernel,
      out_type=jax.ShapeDtypeStruct((SEG_SL, FEMB), jnp.float32),
      mesh=mesh,
      compiler_params=pltpu.CompilerParams(use_tc_tiling_on_sc=False),
      scratch_types=[
          pltpu.VMEM((NCHUNK_SL, ROWS), jnp.int32),
          pltpu.VMEM((2, ROWS, FEMB), jnp.float32),
          pltpu.VMEM((2, G, FEMB), jnp.float32),
          pltpu.SemaphoreType.DMA,
          pltpu.SemaphoreType.DMA,
          pltpu.SemaphoreType.DMA,
          pltpu.SemaphoreType.DMA,
      ],
  )
  def k(idx_hbm, t4, out_hbm, idx_v, rows_v, e_v, g0, g1, w0, w1):
    wid = lax.axis_index("s") * 2 + lax.axis_index("c")
    gsem = [g0, g1]
    wsem = [w0, w1]

    pltpu.sync_copy(idx_hbm.at[wid], idx_v)

    def fire(c, par):
      pltpu.async_copy(t4.at[idx_v.at[c]], rows_v.at[par], gsem[par])

    def wait_gather(c, par):
      pltpu.make_async_copy(
          t4.at[idx_v.at[c]], rows_v.at[par], gsem[par]).wait()

    def wait_write(par):
      pltpu.make_async_copy(
          e_v.at[par], out_hbm.at[pl.ds(0, G)], wsem[par]).wait()

    def sum_chunk(par):
      rows = rows_v.at[par]
      e = e_v.at[par]

      def seg_body(g, _):
        r0 = g * S
        acc = [jnp.zeros((16,), jnp.float32) for _ in range(NLANE)]
        for s in range(S):
          for j in range(NLANE):
            acc[j] = acc[j] + rows[r0 + s, 16 * j:16 * j + 16]
        for j in range(NLANE):
          e[g, 16 * j:16 * j + 16] = acc[j]
        return 0

      lax.fori_loop(0, G, seg_body, 0)

    fire(0, 0)

    def pair_body(cp, _):
      for par in (0, 1):
        c = cp * 2 + par

        @pl.when(c < NCHUNK_SL - 1)
        def _():
          fire(c + 1, 1 - par)

        wait_gather(c, par)

        @pl.when(cp > 0)
        def _():
          wait_write(par)

        sum_chunk(par)
        seg_base = wid * SEG_PER_TILE_SL + c * G
        pltpu.async_copy(
            e_v.at[par], out_hbm.at[pl.ds(seg_base, G)], wsem[par])
      return 0

    lax.fori_loop(0, NCHUNK_SL // 2, pair_body, 0)
    if NCHUNK_SL % 2:
      c = NCHUNK_SL - 1
      wait_gather(c, 0)
      wait_write(0)
      sum_chunk(0)
      seg_base = wid * SEG_PER_TILE_SL + c * G
      pltpu.async_copy(
          e_v.at[0], out_hbm.at[pl.ds(seg_base, G)], wsem[0])
    wait_write(0)
    wait_write(1)

  return k(idx2d, T4)


def _tc_attention(e_all):
  BB = 128

  def body(e_ref, o_ref):
    e = e_ref[...]
    q = jnp.zeros((BB, EMB), jnp.float32)
    o2 = None
    for h in range(HOPS):
      mm = e[:, :, EMB * h:EMB * h + EMB]
      p = jnp.sum(mm * q[:, None, :], axis=2)
      p = p - jnp.max(p, axis=1, keepdims=True)
      a = jnp.exp(p)
      a = a / jnp.sum(a, axis=1, keepdims=True)
      c = e[:, :, EMB * (h + 1):EMB * (h + 1) + EMB]
      o2 = jnp.sum(c * a[:, :, None], axis=1)
      q = q + o2
    o_ref[...] = o2

  return pl.pallas_call(
      body,
      grid=(BSLICE // BB,),
      in_specs=[pl.BlockSpec((BB, M, FEMB), lambda i: (i, 0, 0))],
      out_specs=pl.BlockSpec((BB, EMB), lambda i: (i, 0)),
      out_shape=jax.ShapeDtypeStruct((BSLICE, EMB), jnp.float32),
  )(e_all)


def kernel(context, C0, C1, C2, C3):
  idx_all = context.reshape(NSPLIT, NW, NCHUNK_SL, ROWS).astype(jnp.int32)
  T4 = jnp.concatenate([C0, C1, C2, C3], axis=1)
  outs = []
  for s in range(NSPLIT):
    e4 = _sc_gather_sum(idx_all[s], T4)
    outs.append(_tc_attention(e4.reshape(BSLICE, M, FEMB)))
  return jnp.concatenate(outs, axis=0)
